# Initial kernel scaffold; baseline (speedup 1.0000x reference)
import jax
import jax.numpy as jnp
from jax.experimental import pallas as pl


def kernel(attr_id, attr_category, attr_brand, attr_shop, W_id, W_category, W_brand, W_shop):
    raise NotImplementedError("write your pallas kernel here")


# speedup vs baseline: 6.0994x; 6.0994x over previous
import functools

import jax
import jax.numpy as jnp
from jax import lax
from jax.experimental import pallas as pl
from jax.experimental.pallas import tpu as pltpu
from jax.experimental.pallas import tpu_sc as plsc

_B, _L = 4096, 50
_BL = _B * _L
_NC, _NS, _LANES = 2, 16, 16
_NW = _NC * _NS
_ROWS_PER_W = _BL // _NW
_CHUNK = 128
_NCHUNK = _ROWS_PER_W // _CHUNK
_DIMS = (128, 32, 64, 32)
_OFFS = (0, 128, 160, 224)
_DOUT = 256


def _body(idx_id, idx_cat, idx_br, idx_sh, w_id, w_cat, w_br, w_sh, out,
          idxv, rid, rcat, rbr, rsh, sem):
    wid = lax.axis_index("s") * _NC + lax.axis_index("c")

    pltpu.sync_copy(idx_id.at[wid], idxv.at[0])
    pltpu.sync_copy(idx_cat.at[wid], idxv.at[1])
    pltpu.sync_copy(idx_br.at[wid], idxv.at[2])
    pltpu.sync_copy(idx_sh.at[wid], idxv.at[3])

    def chunk(j, _):
        d0 = pltpu.async_copy(w_id.at[idxv.at[0, j]], rid, sem)
        d1 = pltpu.async_copy(w_cat.at[idxv.at[1, j]], rcat, sem)
        d2 = pltpu.async_copy(w_br.at[idxv.at[2, j]], rbr, sem)
        d3 = pltpu.async_copy(w_sh.at[idxv.at[3, j]], rsh, sem)
        d0.wait()
        d1.wait()
        d2.wait()
        d3.wait()

        for g in range(_CHUNK // _LANES):
            v = idxv[0, j, pl.ds(g * _LANES, _LANES)]
            m = v == 0
            cnt = jnp.sum(jnp.where(m, 1, 0))

            @pl.when(cnt > 0)
            def _():
                rows = g * _LANES + lax.iota(jnp.int32, _LANES)
                zeros = jnp.zeros((_LANES,), jnp.float32)

                def fixcol(c, _):
                    cols = jnp.full((_LANES,), c, jnp.int32)
                    plsc.store_scatter(rid, [rows, cols], zeros, mask=m)
                    return 0

                lax.fori_loop(0, _DIMS[0], fixcol, 0)

        base = wid * _ROWS_PER_W + j * _CHUNK
        pltpu.sync_copy(rid, out.at[pl.ds(base, _CHUNK), pl.ds(_OFFS[0], _DIMS[0])])
        pltpu.sync_copy(rcat, out.at[pl.ds(base, _CHUNK), pl.ds(_OFFS[1], _DIMS[1])])
        pltpu.sync_copy(rbr, out.at[pl.ds(base, _CHUNK), pl.ds(_OFFS[2], _DIMS[2])])
        pltpu.sync_copy(rsh, out.at[pl.ds(base, _CHUNK), pl.ds(_OFFS[3], _DIMS[3])])
        return 0

    lax.fori_loop(0, _NCHUNK, chunk, 0)


_gather = pl.kernel(
    _body,
    out_type=jax.ShapeDtypeStruct((_BL, _DOUT), jnp.float32),
    mesh=plsc.VectorSubcoreMesh(core_axis_name="c", subcore_axis_name="s",
                                num_cores=_NC, num_subcores=_NS),
    scratch_types=[
        pltpu.VMEM((4, _NCHUNK, _CHUNK), jnp.int32),
        pltpu.VMEM((_CHUNK, _DIMS[0]), jnp.float32),
        pltpu.VMEM((_CHUNK, _DIMS[1]), jnp.float32),
        pltpu.VMEM((_CHUNK, _DIMS[2]), jnp.float32),
        pltpu.VMEM((_CHUNK, _DIMS[3]), jnp.float32),
        pltpu.SemaphoreType.DMA,
    ],
    compiler_params=pltpu.CompilerParams(use_tc_tiling_on_sc=False,
                                        needs_layout_passes=False),
)


def kernel(attr_id, attr_category, attr_brand, attr_shop,
           W_id, W_category, W_brand, W_shop):
    shp = (_NW, _NCHUNK, _CHUNK)
    ii = attr_id.astype(jnp.int32).reshape(shp)
    ic = attr_category.astype(jnp.int32).reshape(shp)
    ib = attr_brand.astype(jnp.int32).reshape(shp)
    ish = attr_shop.astype(jnp.int32).reshape(shp)
    out = _gather(ii, ic, ib, ish, W_id, W_category, W_brand, W_shop)
    return out.reshape(_B, _L, _DOUT)


# speedup vs baseline: 6.4841x; 1.0631x over previous
import jax
import jax.numpy as jnp
from jax import lax
from jax.experimental import pallas as pl
from jax.experimental.pallas import tpu as pltpu
from jax.experimental.pallas import tpu_sc as plsc

_B, _L = 4096, 50
_BL = _B * _L
_NC, _NS, _LANES = 2, 16, 16
_NW = _NC * _NS
_ROWS_PER_W = _BL // _NW
_CHUNK = 128
_NCHUNK = _ROWS_PER_W // _CHUNK
_DIMS = (128, 32, 64, 32)
_OFFS = (0, 128, 160, 224)
_DOUT = 256


def _body(idx_id, idx_cat, idx_br, idx_sh, w_id, w_cat, w_br, w_sh, out,
          idxv, rid, rcat, rbr, rsh, gsem, wsem):
    wid = lax.axis_index("s") * _NC + lax.axis_index("c")
    tables = (w_id, w_cat, w_br, w_sh)
    bufs = (rid, rcat, rbr, rsh)

    for t in range(4):
        pltpu.sync_copy((idx_id, idx_cat, idx_br, idx_sh)[t].at[wid],
                        idxv.at[t])

    def issue_gathers(j, b):
        for t in range(4):
            pltpu.async_copy(tables[t].at[idxv.at[t, j]], bufs[t].at[b], gsem)

    def wait_gathers(j, b):
        for t in range(4):
            pltpu.make_async_copy(tables[t].at[idxv.at[t, j]],
                                  bufs[t].at[b], gsem).wait()

    def out_slice(j, t):
        base = wid * _ROWS_PER_W + j * _CHUNK
        return out.at[pl.ds(base, _CHUNK), pl.ds(_OFFS[t], _DIMS[t])]

    def issue_writes(j, b):
        for t in range(4):
            pltpu.async_copy(bufs[t].at[b], out_slice(j, t), wsem)

    def wait_writes(j, b):
        for t in range(4):
            pltpu.make_async_copy(bufs[t].at[b], out_slice(j, t), wsem).wait()

    def fix_padding(j, b):
        for g in range(_CHUNK // _LANES):
            v = idxv[0, j, pl.ds(g * _LANES, _LANES)]
            m = v == 0
            cnt = jnp.sum(jnp.where(m, 1, 0))

            @pl.when(cnt > 0)
            def _():
                rows = g * _LANES + lax.iota(jnp.int32, _LANES)
                zeros = jnp.zeros((_LANES,), jnp.float32)

                def fixcol(c, carry):
                    cols = jnp.full((_LANES,), c, jnp.int32)
                    plsc.store_scatter(rid.at[b], [rows, cols], zeros, mask=m)
                    return carry

                lax.fori_loop(0, _DIMS[0], fixcol, 0)

    issue_gathers(0, 0)

    def chunk(j, carry):
        b = lax.rem(j, 2)
        wait_gathers(j, b)

        @pl.when(j >= 1)
        def _():
            wait_writes(j - 1, 1 - b)

        @pl.when(j + 1 < _NCHUNK)
        def _():
            issue_gathers(j + 1, 1 - b)

        fix_padding(j, b)
        issue_writes(j, b)
        return carry

    lax.fori_loop(0, _NCHUNK, chunk, 0)
    wait_writes(_NCHUNK - 1, (_NCHUNK - 1) % 2)


_gather = pl.kernel(
    _body,
    out_type=jax.ShapeDtypeStruct((_BL, _DOUT), jnp.float32),
    mesh=plsc.VectorSubcoreMesh(core_axis_name="c", subcore_axis_name="s",
                                num_cores=_NC, num_subcores=_NS),
    scratch_types=[
        pltpu.VMEM((4, _NCHUNK, _CHUNK), jnp.int32),
        pltpu.VMEM((2, _CHUNK, _DIMS[0]), jnp.float32),
        pltpu.VMEM((2, _CHUNK, _DIMS[1]), jnp.float32),
        pltpu.VMEM((2, _CHUNK, _DIMS[2]), jnp.float32),
        pltpu.VMEM((2, _CHUNK, _DIMS[3]), jnp.float32),
        pltpu.SemaphoreType.DMA,
        pltpu.SemaphoreType.DMA,
    ],
    compiler_params=pltpu.CompilerParams(use_tc_tiling_on_sc=False,
                                         needs_layout_passes=False),
)


def kernel(attr_id, attr_category, attr_brand, attr_shop,
           W_id, W_category, W_brand, W_shop):
    shp = (_NW, _NCHUNK, _CHUNK)
    ii = attr_id.astype(jnp.int32).reshape(shp)
    ic = attr_category.astype(jnp.int32).reshape(shp)
    ib = attr_brand.astype(jnp.int32).reshape(shp)
    ish = attr_shop.astype(jnp.int32).reshape(shp)
    out = _gather(ii, ic, ib, ish, W_id, W_category, W_brand, W_shop)
    return out.reshape(_B, _L, _DOUT)
